# baseline (speedup 1.0000x reference)
import jax
import jax.numpy as jnp
from jax.experimental import pallas as pl
from jax.experimental.pallas import tpu as pltpu


def _ffn_kernel(x_ref, w1_ref, b1_ref, w2_ref, b2_ref, o_ref, xb_ref):
    c = pl.program_id(0)

    @pl.when(c == 0)
    def _():
        xb_ref[...] = x_ref[...].astype(jnp.bfloat16)

    w1b = w1_ref[...].astype(jnp.bfloat16)
    h = jnp.dot(xb_ref[...], w1b, preferred_element_type=jnp.float32)
    h = jax.nn.gelu(h + b1_ref[...], approximate=True)
    w2b = w2_ref[...].astype(jnp.bfloat16)
    y = jnp.dot(h.astype(jnp.bfloat16), w2b,
                preferred_element_type=jnp.float32)

    @pl.when(c == 0)
    def _():
        o_ref[...] = y + b2_ref[...]

    @pl.when(c != 0)
    def _():
        o_ref[...] = o_ref[...] + y


def kernel(x, w1, b1, w2, b2):
    b, n, d = x.shape
    dh = w1.shape[1]
    m = b * n
    x2 = x.reshape(m, d)

    nc = 4 if dh % (4 * 256) == 0 else 1
    ch = dh // nc
    cost = pl.CostEstimate(
        flops=4 * m * d * dh,
        transcendentals=m * dh,
        bytes_accessed=(m * d * 2 + 2 * d * dh + d + dh) * 4,
    )
    out = pl.pallas_call(
        _ffn_kernel,
        out_shape=jax.ShapeDtypeStruct((m, d), x.dtype),
        grid_spec=pltpu.PrefetchScalarGridSpec(
            num_scalar_prefetch=0,
            grid=(nc,),
            in_specs=[
                pl.BlockSpec((m, d), lambda c: (0, 0)),
                pl.BlockSpec((d, ch), lambda c: (0, c)),
                pl.BlockSpec((1, ch), lambda c: (0, c)),
                pl.BlockSpec((ch, d), lambda c: (c, 0)),
                pl.BlockSpec((1, d), lambda c: (0, 0)),
            ],
            out_specs=pl.BlockSpec((m, d), lambda c: (0, 0)),
            scratch_shapes=[
                pltpu.VMEM((m, d), jnp.bfloat16),
            ],
        ),
        compiler_params=pltpu.CompilerParams(
            dimension_semantics=("arbitrary",),
            vmem_limit_bytes=100 * 1024 * 1024,
        ),
        cost_estimate=cost,
    )(x2, w1, b1, w2, b2)
    return out.reshape(b, n, d)


# speedup vs baseline: 1.2084x; 1.2084x over previous
import jax
import jax.numpy as jnp
from jax.experimental import pallas as pl
from jax.experimental.pallas import tpu as pltpu


def _ffn_kernel(x_ref, w1_ref, b1_ref, w2_ref, b2_ref, o_ref, w1s_ref):
    @pl.when(pl.program_id(0) == 0)
    def _():
        w1s_ref[...] = w1_ref[...].astype(jnp.bfloat16)

    xb = x_ref[...].astype(jnp.bfloat16)
    h = jnp.dot(xb, w1s_ref[...], preferred_element_type=jnp.float32)
    h = jax.nn.gelu(h + b1_ref[...], approximate=True)
    y = jnp.dot(h.astype(jnp.bfloat16), w2_ref[...].astype(jnp.bfloat16),
                preferred_element_type=jnp.float32)
    o_ref[...] = y + b2_ref[...]


def _row_tile(m, target):
    if m % target == 0:
        return target
    t = (min(m, target) // 8) * 8
    while t >= 8:
        if m % t == 0:
            return t
        t -= 8
    return m


def kernel(x, w1, b1, w2, b2):
    b, n, d = x.shape
    dh = w1.shape[1]
    m = b * n
    x2 = x.reshape(m, d)

    tm = _row_tile(m, 1024)
    nin = m // tm
    cost = pl.CostEstimate(
        flops=4 * m * d * dh,
        transcendentals=m * dh,
        bytes_accessed=(m * d * 2 + 2 * d * dh + d + dh) * 4,
    )
    out = pl.pallas_call(
        _ffn_kernel,
        out_shape=jax.ShapeDtypeStruct((m, d), x.dtype),
        grid_spec=pltpu.PrefetchScalarGridSpec(
            num_scalar_prefetch=0,
            grid=(nin,),
            in_specs=[
                pl.BlockSpec((tm, d), lambda j: (j, 0)),
                pl.BlockSpec((d, dh), lambda j: (0, 0)),
                pl.BlockSpec((1, dh), lambda j: (0, 0)),
                pl.BlockSpec((dh, d), lambda j: (0, 0)),
                pl.BlockSpec((1, d), lambda j: (0, 0)),
            ],
            out_specs=pl.BlockSpec((tm, d), lambda j: (j, 0)),
            scratch_shapes=[
                pltpu.VMEM((d, dh), jnp.bfloat16),
            ],
        ),
        compiler_params=pltpu.CompilerParams(
            dimension_semantics=("arbitrary",),
            vmem_limit_bytes=100 * 1024 * 1024,
        ),
        cost_estimate=cost,
    )(x2, w1, b1, w2, b2)
    return out.reshape(b, n, d)


# speedup vs baseline: 1.2216x; 1.0109x over previous
import jax
import jax.numpy as jnp
from jax.experimental import pallas as pl
from jax.experimental.pallas import tpu as pltpu


def _ffn_kernel(x_ref, w1_ref, b1_ref, w2_ref, b2_ref, o_ref):
    xb = x_ref[...].astype(jnp.bfloat16)
    h = jnp.dot(xb, w1_ref[...].astype(jnp.bfloat16),
                preferred_element_type=jnp.float32)
    h = jax.nn.gelu(h + b1_ref[...], approximate=True)
    y = jnp.dot(h.astype(jnp.bfloat16), w2_ref[...].astype(jnp.bfloat16),
                preferred_element_type=jnp.float32)
    o_ref[...] = y + b2_ref[...]


def _row_tile(m, target):
    if m % target == 0:
        return target
    t = (min(m, target) // 8) * 8
    while t >= 8:
        if m % t == 0:
            return t
        t -= 8
    return m


def kernel(x, w1, b1, w2, b2):
    b, n, d = x.shape
    dh = w1.shape[1]
    m = b * n
    x2 = x.reshape(m, d)

    tm = _row_tile(m, 1024)
    nin = m // tm
    cost = pl.CostEstimate(
        flops=4 * m * d * dh,
        transcendentals=m * dh,
        bytes_accessed=(m * d * 2 + 2 * d * dh + d + dh) * 4,
    )
    out = pl.pallas_call(
        _ffn_kernel,
        out_shape=jax.ShapeDtypeStruct((m, d), x.dtype),
        grid_spec=pltpu.PrefetchScalarGridSpec(
            num_scalar_prefetch=0,
            grid=(nin,),
            in_specs=[
                pl.BlockSpec((tm, d), lambda j: (j, 0)),
                pl.BlockSpec((d, dh), lambda j: (0, 0)),
                pl.BlockSpec((1, dh), lambda j: (0, 0)),
                pl.BlockSpec((dh, d), lambda j: (0, 0)),
                pl.BlockSpec((1, d), lambda j: (0, 0)),
            ],
            out_specs=pl.BlockSpec((tm, d), lambda j: (j, 0)),
        ),
        compiler_params=pltpu.CompilerParams(
            dimension_semantics=("arbitrary",),
            vmem_limit_bytes=100 * 1024 * 1024,
        ),
        cost_estimate=cost,
    )(x2, w1, b1, w2, b2)
    return out.reshape(b, n, d)
